# baseline (speedup 1.0000x reference)
import jax
import jax.numpy as jnp
from jax.experimental import pallas as pl
from jax.experimental.pallas import tpu as pltpu

B = 16
CLS = 100
POOL = 100
CTX_LEN = 12
D = 512
SEQ = 77
SUF = SEQ - 1 - CTX_LEN * 2
NC_SUF = SEQ - 1 - CTX_LEN
NC_SEQ = 1 + 2 * CTX_LEN + NC_SUF

CB = 25
NCB = CLS // CB

ROW = CTX_LEN * D
PRE_W = D
CTX_W = 2 * CTX_LEN * D
SUF_W = SUF * D
OUT_W = SEQ * D


def _prompts_body(ig_ref, ia_ref, g0, g1, a0, a1, pre, suf, tokp, out, tok_out):
    b = pl.program_id(1)
    use_g = b < 8
    row0 = jnp.where(use_g, g0[...], a0[...])
    row1 = jnp.where(use_g, g1[...], a1[...])
    out[:, :, 0:PRE_W] = pre[...]
    ctx = jnp.concatenate([row0, row1], axis=2)
    out[:, :, PRE_W:PRE_W + CTX_W] = jnp.broadcast_to(ctx, (1, CB, CTX_W))
    out[:, :, PRE_W + CTX_W:OUT_W] = suf[...]
    tok_out[...] = tokp[...]


def _build_prompts_call():
    def g0_map(cb, b, ig, ia):
        return jnp.where(b < 8, ig[2 * b], 0), 0, 0

    def g1_map(cb, b, ig, ia):
        return jnp.where(b < 8, ig[2 * b + 1], 0), 0, 0

    def a0_map(cb, b, ig, ia):
        return jnp.where(b >= 8, ia[(2 * b - 16) % B], 0), 0, 0

    def a1_map(cb, b, ig, ia):
        return jnp.where(b >= 8, ia[(2 * b - 15) % B], 0), 0, 0

    grid_spec = pltpu.PrefetchScalarGridSpec(
        num_scalar_prefetch=2,
        grid=(NCB, B),
        in_specs=[
            pl.BlockSpec((1, 1, ROW), g0_map),
            pl.BlockSpec((1, 1, ROW), g1_map),
            pl.BlockSpec((1, 1, ROW), a0_map),
            pl.BlockSpec((1, 1, ROW), a1_map),
            pl.BlockSpec((1, CB, PRE_W), lambda cb, b, ig, ia: (cb, 0, 0)),
            pl.BlockSpec((1, CB, SUF_W), lambda cb, b, ig, ia: (cb, 0, 0)),
            pl.BlockSpec((1, CLS, SEQ), lambda cb, b, ig, ia: (0, 0, 0)),
        ],
        out_specs=[
            pl.BlockSpec((1, CB, OUT_W),
                         lambda cb, b, ig, ia: (b * NCB + cb, 0, 0)),
            pl.BlockSpec((1, CLS, SEQ), lambda cb, b, ig, ia: (b, 0, 0)),
        ],
    )
    return pl.pallas_call(
        _prompts_body,
        grid_spec=grid_spec,
        out_shape=[
            jax.ShapeDtypeStruct((B * NCB, CB, OUT_W), jnp.float32),
            jax.ShapeDtypeStruct((B, CLS, SEQ), jnp.int32),
        ],
    )


def _nc_body(ncp, gp, ap, ncs, nctok, out, tok_out):
    out[:, :, 0:D] = jnp.broadcast_to(ncp[...], (1, CB, D))
    out[:, :, D:D + ROW] = gp[...]
    out[:, :, D + ROW:D + 2 * ROW] = ap[...]
    out[:, :, D + 2 * ROW:NC_SEQ * D] = jnp.broadcast_to(
        ncs[...], (1, CB, NC_SUF * D))
    tok_out[...] = jnp.broadcast_to(nctok[...], (1, POOL, SEQ))


def _build_nc_call():
    return pl.pallas_call(
        _nc_body,
        grid=(NCB,),
        in_specs=[
            pl.BlockSpec((1, 1, D), lambda i: (0, 0, 0)),
            pl.BlockSpec((1, CB, ROW), lambda i: (i, 0, 0)),
            pl.BlockSpec((1, CB, ROW), lambda i: (i, 0, 0)),
            pl.BlockSpec((1, 1, NC_SUF * D), lambda i: (0, 0, 0)),
            pl.BlockSpec((1, 1, SEQ), lambda i: (0, 0, 0)),
        ],
        out_specs=[
            pl.BlockSpec((1, CB, NC_SEQ * D), lambda i: (i, 0, 0)),
            pl.BlockSpec((1, POOL, SEQ), lambda i: (0, 0, 0)),
        ],
        out_shape=[
            jax.ShapeDtypeStruct((NCB, CB, NC_SEQ * D), jnp.float32),
            jax.ShapeDtypeStruct((1, POOL, SEQ), jnp.int32),
        ],
    )


def kernel(indices_g, indices_a, global_prompt, attribute_prompt,
           token_prefix, token_suffix, nc_token_prefix, nc_token_suffix,
           tokenized_prompts, nc_tokenized_prompts):
    ig = indices_g.astype(jnp.int32)
    ia = indices_a.astype(jnp.int32)
    gp3 = global_prompt.reshape(POOL, 1, ROW)
    ap3 = attribute_prompt.reshape(POOL, 1, ROW)
    pre3 = token_prefix.reshape(NCB, CB, PRE_W)
    suf3 = token_suffix.reshape(NCB, CB, SUF_W)
    tokp3 = tokenized_prompts.astype(jnp.int32).reshape(1, CLS, SEQ)

    prompts3, tok3 = _build_prompts_call()(
        ig, ia, gp3, gp3, ap3, ap3, pre3, suf3, tokp3)

    gpn = global_prompt.reshape(NCB, CB, ROW)
    apn = attribute_prompt.reshape(NCB, CB, ROW)
    ncp3 = nc_token_prefix.reshape(1, 1, D)
    ncs3 = nc_token_suffix.reshape(1, 1, NC_SUF * D)
    nctok3 = nc_tokenized_prompts.astype(jnp.int32).reshape(1, 1, SEQ)
    nc_prompts3, nc_tok3 = _build_nc_call()(ncp3, gpn, apn, ncs3, nctok3)

    return (prompts3.reshape(B * CLS, SEQ, D),
            tok3.reshape(B * CLS, SEQ),
            nc_prompts3.reshape(POOL, NC_SEQ, D),
            nc_tok3.reshape(POOL, SEQ))


# speedup vs baseline: 2.3017x; 2.3017x over previous
import jax
import jax.numpy as jnp
from jax.experimental import pallas as pl
from jax.experimental.pallas import tpu as pltpu

B = 16
CLS = 100
POOL = 100
CTX_LEN = 12
D = 512
SEQ = 77
SUF = SEQ - 1 - CTX_LEN * 2
NC_SUF = SEQ - 1 - CTX_LEN
NC_SEQ = 1 + 2 * CTX_LEN + NC_SUF

CB = 25
NCB = CLS // CB


def _prompts_body(ig_ref, ia_ref, g0, g1, a0, a1, pre, suf, out):
    b = pl.program_id(1)
    use_g = b < 8
    row0 = jnp.where(use_g, g0[...], a0[...])
    row1 = jnp.where(use_g, g1[...], a1[...])
    out[:, 0:1, :] = pre[...]
    out[:, 1:1 + CTX_LEN, :] = jnp.broadcast_to(row0, (CB, CTX_LEN, D))
    out[:, 1 + CTX_LEN:1 + 2 * CTX_LEN, :] = jnp.broadcast_to(
        row1, (CB, CTX_LEN, D))
    out[:, 1 + 2 * CTX_LEN:SEQ, :] = suf[...]


def _build_prompts_call():
    def g0_map(cb, b, ig, ia):
        return jnp.where(b < 8, ig[2 * b], 0), 0, 0

    def g1_map(cb, b, ig, ia):
        return jnp.where(b < 8, ig[2 * b + 1], 0), 0, 0

    def a0_map(cb, b, ig, ia):
        return jnp.where(b >= 8, ia[(2 * b - 16) % B], 0), 0, 0

    def a1_map(cb, b, ig, ia):
        return jnp.where(b >= 8, ia[(2 * b - 15) % B], 0), 0, 0

    grid_spec = pltpu.PrefetchScalarGridSpec(
        num_scalar_prefetch=2,
        grid=(NCB, B),
        in_specs=[
            pl.BlockSpec((1, CTX_LEN, D), g0_map),
            pl.BlockSpec((1, CTX_LEN, D), g1_map),
            pl.BlockSpec((1, CTX_LEN, D), a0_map),
            pl.BlockSpec((1, CTX_LEN, D), a1_map),
            pl.BlockSpec((CB, 1, D), lambda cb, b, ig, ia: (cb, 0, 0)),
            pl.BlockSpec((CB, SUF, D), lambda cb, b, ig, ia: (cb, 0, 0)),
        ],
        out_specs=pl.BlockSpec((CB, SEQ, D),
                               lambda cb, b, ig, ia: (b * NCB + cb, 0, 0)),
    )
    return pl.pallas_call(
        _prompts_body,
        grid_spec=grid_spec,
        out_shape=jax.ShapeDtypeStruct((B * CLS, SEQ, D), jnp.float32),
    )


def _nc_body(ncp, gp, ap, ncs, nctok, tokp, out, nc_tok_out, tok_out):
    out[:, 0:1, :] = jnp.broadcast_to(ncp[...], (CB, 1, D))
    out[:, 1:1 + CTX_LEN, :] = gp[...]
    out[:, 1 + CTX_LEN:1 + 2 * CTX_LEN, :] = ap[...]
    out[:, 1 + 2 * CTX_LEN:NC_SEQ, :] = jnp.broadcast_to(
        ncs[...], (CB, NC_SUF, D))
    nc_tok_out[...] = jnp.broadcast_to(nctok[...], (POOL, SEQ))
    t = tokp[...]
    for b in range(B):
        tok_out[pl.ds(b * CLS, CLS), :] = t


def _build_nc_call():
    return pl.pallas_call(
        _nc_body,
        grid=(NCB,),
        in_specs=[
            pl.BlockSpec((1, 1, D), lambda i: (0, 0, 0)),
            pl.BlockSpec((CB, CTX_LEN, D), lambda i: (i, 0, 0)),
            pl.BlockSpec((CB, CTX_LEN, D), lambda i: (i, 0, 0)),
            pl.BlockSpec((1, NC_SUF, D), lambda i: (0, 0, 0)),
            pl.BlockSpec((1, SEQ), lambda i: (0, 0)),
            pl.BlockSpec((CLS, SEQ), lambda i: (0, 0)),
        ],
        out_specs=[
            pl.BlockSpec((CB, NC_SEQ, D), lambda i: (i, 0, 0)),
            pl.BlockSpec((POOL, SEQ), lambda i: (0, 0)),
            pl.BlockSpec((B * CLS, SEQ), lambda i: (0, 0)),
        ],
        out_shape=[
            jax.ShapeDtypeStruct((POOL, NC_SEQ, D), jnp.float32),
            jax.ShapeDtypeStruct((POOL, SEQ), jnp.int32),
            jax.ShapeDtypeStruct((B * CLS, SEQ), jnp.int32),
        ],
    )


def kernel(indices_g, indices_a, global_prompt, attribute_prompt,
           token_prefix, token_suffix, nc_token_prefix, nc_token_suffix,
           tokenized_prompts, nc_tokenized_prompts):
    ig = indices_g.astype(jnp.int32)
    ia = indices_a.astype(jnp.int32)
    tokp = tokenized_prompts.astype(jnp.int32)
    nctok = nc_tokenized_prompts.astype(jnp.int32)

    prompts = _build_prompts_call()(
        ig, ia, global_prompt, global_prompt, attribute_prompt,
        attribute_prompt, token_prefix, token_suffix)

    nc_prompts, nc_tok, tok = _build_nc_call()(
        nc_token_prefix, global_prompt, attribute_prompt,
        nc_token_suffix, nctok, tokp)

    return (prompts, tok, nc_prompts, nc_tok)


# speedup vs baseline: 2.3378x; 1.0157x over previous
import jax
import jax.numpy as jnp
from jax.experimental import pallas as pl
from jax.experimental.pallas import tpu as pltpu

B = 16
CLS = 100
POOL = 100
CTX_LEN = 12
D = 512
SEQ = 77
SUF = SEQ - 1 - CTX_LEN * 2
NC_SUF = SEQ - 1 - CTX_LEN
NC_SEQ = 1 + 2 * CTX_LEN + NC_SUF

CB = 25
NCB = CLS // CB


def _prompts_body(ig_ref, ia_ref, g0, g1, a0, a1, pre, suf, out):
    s = pl.program_id(0)
    b = s // NCB
    cb = s % NCB
    use_g = b < 8
    row0 = jnp.where(use_g, g0[...], a0[...])
    row1 = jnp.where(use_g, g1[...], a1[...])
    out[:, 0:1, :] = pre[...]
    out[:, 1:1 + CTX_LEN, :] = jnp.broadcast_to(row0, (CB, CTX_LEN, D))
    out[:, 1 + CTX_LEN:1 + 2 * CTX_LEN, :] = jnp.broadcast_to(
        row1, (CB, CTX_LEN, D))
    out[:, 1 + 2 * CTX_LEN:SEQ, :] = suf[pl.ds(cb * CB, CB)]


def _build_prompts_call():
    def g0_map(s, ig, ia):
        b = s // NCB
        return jnp.where(b < 8, ig[2 * b], 0), 0, 0

    def g1_map(s, ig, ia):
        b = s // NCB
        return jnp.where(b < 8, ig[2 * b + 1], 0), 0, 0

    def a0_map(s, ig, ia):
        b = s // NCB
        return jnp.where(b >= 8, ia[(2 * b - 16) % B], 0), 0, 0

    def a1_map(s, ig, ia):
        b = s // NCB
        return jnp.where(b >= 8, ia[(2 * b - 15) % B], 0), 0, 0

    grid_spec = pltpu.PrefetchScalarGridSpec(
        num_scalar_prefetch=2,
        grid=(B * NCB,),
        in_specs=[
            pl.BlockSpec((1, CTX_LEN, D), g0_map),
            pl.BlockSpec((1, CTX_LEN, D), g1_map),
            pl.BlockSpec((1, CTX_LEN, D), a0_map),
            pl.BlockSpec((1, CTX_LEN, D), a1_map),
            pl.BlockSpec((CB, 1, D), lambda s, ig, ia: (s % NCB, 0, 0)),
            pl.BlockSpec((CLS, SUF, D), lambda s, ig, ia: (0, 0, 0)),
        ],
        out_specs=pl.BlockSpec((CB, SEQ, D),
                               lambda s, ig, ia: (s, 0, 0)),
    )
    return pl.pallas_call(
        _prompts_body,
        grid_spec=grid_spec,
        out_shape=jax.ShapeDtypeStruct((B * CLS, SEQ, D), jnp.float32),
    )


def _nc_body(ncp, gp, ap, ncs, nctok, tokp, out, nc_tok_out, tok_out):
    out[:, 0:1, :] = jnp.broadcast_to(ncp[...], (CB, 1, D))
    out[:, 1:1 + CTX_LEN, :] = gp[...]
    out[:, 1 + CTX_LEN:1 + 2 * CTX_LEN, :] = ap[...]
    out[:, 1 + 2 * CTX_LEN:NC_SEQ, :] = jnp.broadcast_to(
        ncs[...], (CB, NC_SUF, D))
    nc_tok_out[...] = jnp.broadcast_to(nctok[...], (POOL, SEQ))
    t = tokp[...]
    for b in range(B):
        tok_out[pl.ds(b * CLS, CLS), :] = t


def _build_nc_call():
    return pl.pallas_call(
        _nc_body,
        grid=(NCB,),
        in_specs=[
            pl.BlockSpec((1, 1, D), lambda i: (0, 0, 0)),
            pl.BlockSpec((CB, CTX_LEN, D), lambda i: (i, 0, 0)),
            pl.BlockSpec((CB, CTX_LEN, D), lambda i: (i, 0, 0)),
            pl.BlockSpec((1, NC_SUF, D), lambda i: (0, 0, 0)),
            pl.BlockSpec((1, SEQ), lambda i: (0, 0)),
            pl.BlockSpec((CLS, SEQ), lambda i: (0, 0)),
        ],
        out_specs=[
            pl.BlockSpec((CB, NC_SEQ, D), lambda i: (i, 0, 0)),
            pl.BlockSpec((POOL, SEQ), lambda i: (0, 0)),
            pl.BlockSpec((B * CLS, SEQ), lambda i: (0, 0)),
        ],
        out_shape=[
            jax.ShapeDtypeStruct((POOL, NC_SEQ, D), jnp.float32),
            jax.ShapeDtypeStruct((POOL, SEQ), jnp.int32),
            jax.ShapeDtypeStruct((B * CLS, SEQ), jnp.int32),
        ],
    )


def kernel(indices_g, indices_a, global_prompt, attribute_prompt,
           token_prefix, token_suffix, nc_token_prefix, nc_token_suffix,
           tokenized_prompts, nc_tokenized_prompts):
    ig = indices_g.astype(jnp.int32)
    ia = indices_a.astype(jnp.int32)
    tokp = tokenized_prompts.astype(jnp.int32)
    nctok = nc_tokenized_prompts.astype(jnp.int32)

    prompts = _build_prompts_call()(
        ig, ia, global_prompt, global_prompt, attribute_prompt,
        attribute_prompt, token_prefix, token_suffix)

    nc_prompts, nc_tok, tok = _build_nc_call()(
        nc_token_prefix, global_prompt, attribute_prompt,
        nc_token_suffix, nctok, tokp)

    return (prompts, tok, nc_prompts, nc_tok)


# speedup vs baseline: 2.3771x; 1.0168x over previous
import jax
import jax.numpy as jnp
from jax.experimental import pallas as pl
from jax.experimental.pallas import tpu as pltpu

B = 16
CLS = 100
POOL = 100
CTX_LEN = 12
D = 512
SEQ = 77
SUF = SEQ - 1 - CTX_LEN * 2
NC_SUF = SEQ - 1 - CTX_LEN
NC_SEQ = 1 + 2 * CTX_LEN + NC_SUF

CB = 50
NCB = CLS // CB


def _prompts_body(ig_ref, ia_ref, g0, g1, a0, a1, pre, suf, out):
    s = pl.program_id(0)
    b = s // NCB
    cb = s % NCB
    use_g = b < 8
    row0 = jnp.where(use_g, g0[...], a0[...])
    row1 = jnp.where(use_g, g1[...], a1[...])
    out[:, 0:1, :] = pre[...]
    out[:, 1:1 + CTX_LEN, :] = jnp.broadcast_to(row0, (CB, CTX_LEN, D))
    out[:, 1 + CTX_LEN:1 + 2 * CTX_LEN, :] = jnp.broadcast_to(
        row1, (CB, CTX_LEN, D))
    out[:, 1 + 2 * CTX_LEN:SEQ, :] = suf[pl.ds(cb * CB, CB)]


def _build_prompts_call():
    def g0_map(s, ig, ia):
        b = s // NCB
        return jnp.where(b < 8, ig[2 * b], 0), 0, 0

    def g1_map(s, ig, ia):
        b = s // NCB
        return jnp.where(b < 8, ig[2 * b + 1], 0), 0, 0

    def a0_map(s, ig, ia):
        b = s // NCB
        return jnp.where(b >= 8, ia[(2 * b - 16) % B], 0), 0, 0

    def a1_map(s, ig, ia):
        b = s // NCB
        return jnp.where(b >= 8, ia[(2 * b - 15) % B], 0), 0, 0

    grid_spec = pltpu.PrefetchScalarGridSpec(
        num_scalar_prefetch=2,
        grid=(B * NCB,),
        in_specs=[
            pl.BlockSpec((1, CTX_LEN, D), g0_map),
            pl.BlockSpec((1, CTX_LEN, D), g1_map),
            pl.BlockSpec((1, CTX_LEN, D), a0_map),
            pl.BlockSpec((1, CTX_LEN, D), a1_map),
            pl.BlockSpec((CB, 1, D), lambda s, ig, ia: (s % NCB, 0, 0)),
            pl.BlockSpec((CLS, SUF, D), lambda s, ig, ia: (0, 0, 0)),
        ],
        out_specs=pl.BlockSpec((CB, SEQ, D),
                               lambda s, ig, ia: (s, 0, 0)),
    )
    return pl.pallas_call(
        _prompts_body,
        grid_spec=grid_spec,
        out_shape=jax.ShapeDtypeStruct((B * CLS, SEQ, D), jnp.float32),
    )


def _nc_body(ncp, gp, ap, ncs, nctok, tokp, out, nc_tok_out, tok_out):
    out[:, 0:1, :] = jnp.broadcast_to(ncp[...], (CB, 1, D))
    out[:, 1:1 + CTX_LEN, :] = gp[...]
    out[:, 1 + CTX_LEN:1 + 2 * CTX_LEN, :] = ap[...]
    out[:, 1 + 2 * CTX_LEN:NC_SEQ, :] = jnp.broadcast_to(
        ncs[...], (CB, NC_SUF, D))
    nc_tok_out[...] = jnp.broadcast_to(nctok[...], (POOL, SEQ))
    t = tokp[...]
    for b in range(B):
        tok_out[pl.ds(b * CLS, CLS), :] = t


def _build_nc_call():
    return pl.pallas_call(
        _nc_body,
        grid=(NCB,),
        in_specs=[
            pl.BlockSpec((1, 1, D), lambda i: (0, 0, 0)),
            pl.BlockSpec((CB, CTX_LEN, D), lambda i: (i, 0, 0)),
            pl.BlockSpec((CB, CTX_LEN, D), lambda i: (i, 0, 0)),
            pl.BlockSpec((1, NC_SUF, D), lambda i: (0, 0, 0)),
            pl.BlockSpec((1, SEQ), lambda i: (0, 0)),
            pl.BlockSpec((CLS, SEQ), lambda i: (0, 0)),
        ],
        out_specs=[
            pl.BlockSpec((CB, NC_SEQ, D), lambda i: (i, 0, 0)),
            pl.BlockSpec((POOL, SEQ), lambda i: (0, 0)),
            pl.BlockSpec((B * CLS, SEQ), lambda i: (0, 0)),
        ],
        out_shape=[
            jax.ShapeDtypeStruct((POOL, NC_SEQ, D), jnp.float32),
            jax.ShapeDtypeStruct((POOL, SEQ), jnp.int32),
            jax.ShapeDtypeStruct((B * CLS, SEQ), jnp.int32),
        ],
    )


def kernel(indices_g, indices_a, global_prompt, attribute_prompt,
           token_prefix, token_suffix, nc_token_prefix, nc_token_suffix,
           tokenized_prompts, nc_tokenized_prompts):
    ig = indices_g.astype(jnp.int32)
    ia = indices_a.astype(jnp.int32)
    tokp = tokenized_prompts.astype(jnp.int32)
    nctok = nc_tokenized_prompts.astype(jnp.int32)

    prompts = _build_prompts_call()(
        ig, ia, global_prompt, global_prompt, attribute_prompt,
        attribute_prompt, token_prefix, token_suffix)

    nc_prompts, nc_tok, tok = _build_nc_call()(
        nc_token_prefix, global_prompt, attribute_prompt,
        nc_token_suffix, nctok, tokp)

    return (prompts, tok, nc_prompts, nc_tok)
